# baseline (speedup 1.0000x reference)
import jax
import jax.numpy as jnp
from jax.experimental import pallas as pl
from jax.experimental.pallas import tpu as pltpu

_STRIPE_ROWS = (56, 40, 24, 8)


def _dma_copy(in_hbm, out_hbm, *refs):
    n = len(_STRIPE_ROWS)
    bufs = refs[:n]
    in_sems, out_sems = refs[n], refs[n + 1]
    offs = [sum(_STRIPE_ROWS[:k]) for k in range(n)]

    def read_copy(k):
        return pltpu.make_async_copy(
            in_hbm.at[pl.ds(offs[k], _STRIPE_ROWS[k]), :], bufs[k], in_sems.at[k]
        )

    def write_copy(k):
        return pltpu.make_async_copy(
            bufs[k], out_hbm.at[pl.ds(offs[k], _STRIPE_ROWS[k]), :], out_sems.at[k]
        )

    for k in range(n):
        read_copy(k).start()
    for k in range(n):
        read_copy(k).wait()
        write_copy(k).start()
    for k in range(n):
        write_copy(k).wait()


def kernel(features):
    rows, cols = features.shape
    n = len(_STRIPE_ROWS)
    return pl.pallas_call(
        _dma_copy,
        in_specs=[pl.BlockSpec(memory_space=pltpu.MemorySpace.HBM)],
        out_specs=pl.BlockSpec(memory_space=pltpu.MemorySpace.HBM),
        out_shape=jax.ShapeDtypeStruct((rows, cols), features.dtype),
        scratch_shapes=[pltpu.VMEM((r, cols), features.dtype) for r in _STRIPE_ROWS]
        + [
            pltpu.SemaphoreType.DMA((n,)),
            pltpu.SemaphoreType.DMA((n,)),
        ],
    )(features)


# speedup vs baseline: 1.0243x; 1.0243x over previous
import jax
import jax.numpy as jnp
from jax.experimental import pallas as pl
from jax.experimental.pallas import tpu as pltpu

_STRIPE_ROWS = (48, 40, 24, 16)


def _dma_copy(in_hbm, out_hbm, *refs):
    n = len(_STRIPE_ROWS)
    bufs = refs[:n]
    in_sems, out_sems = refs[n], refs[n + 1]
    offs = [sum(_STRIPE_ROWS[:k]) for k in range(n)]

    def read_copy(k):
        return pltpu.make_async_copy(
            in_hbm.at[pl.ds(offs[k], _STRIPE_ROWS[k]), :], bufs[k], in_sems.at[k]
        )

    def write_copy(k):
        return pltpu.make_async_copy(
            bufs[k], out_hbm.at[pl.ds(offs[k], _STRIPE_ROWS[k]), :], out_sems.at[k]
        )

    for k in range(n):
        read_copy(k).start()
    for k in range(n):
        read_copy(k).wait()
        write_copy(k).start()
    for k in range(n):
        write_copy(k).wait()


def kernel(features):
    rows, cols = features.shape
    n = len(_STRIPE_ROWS)
    return pl.pallas_call(
        _dma_copy,
        in_specs=[pl.BlockSpec(memory_space=pltpu.MemorySpace.HBM)],
        out_specs=pl.BlockSpec(memory_space=pltpu.MemorySpace.HBM),
        out_shape=jax.ShapeDtypeStruct((rows, cols), features.dtype),
        scratch_shapes=[pltpu.VMEM((r, cols), features.dtype) for r in _STRIPE_ROWS]
        + [
            pltpu.SemaphoreType.DMA((n,)),
            pltpu.SemaphoreType.DMA((n,)),
        ],
    )(features)
